# baseline (speedup 1.0000x reference)
import jax
import jax.numpy as jnp
from jax.experimental import pallas as pl

N = 50000
G = 128


def _bn(x, g, b):
    m = x.mean(axis=0)
    v = x.var(axis=0)
    return (x - m) / jnp.sqrt(v + 1e-5) * g + b


def _gcn(x, edge_index, W, b):
    src, dst = edge_index[0], edge_index[1]
    n = x.shape[0]
    deg = jax.ops.segment_sum(jnp.ones(src.shape[0], dtype=x.dtype), dst, num_segments=n) + 1.0
    dinv = 1.0 / jnp.sqrt(deg)
    h = x @ W
    coef = (dinv[src] * dinv[dst])[:, None]
    agg = jax.ops.segment_sum(h[src] * coef, dst, num_segments=n)
    agg = agg + h * (1.0 / deg)[:, None]
    return agg + b


def _gcn_stack(x, edge_index, batch, params, pre):
    for i in range(4):
        x = jax.nn.relu(_gcn(x, edge_index, params[pre + '_W' + str(i)], params[pre + '_b' + str(i)]))
        x = _bn(x, params[pre + '_g' + str(i)], params[pre + '_be' + str(i)])
    return jax.ops.segment_sum(x, batch, num_segments=G)


def _copy_kernel(x_ref, o_ref):
    o_ref[...] = x_ref[...]


def kernel(solute_x, solute_edge_index, solute_batch, solvent_x, solvent_edge_index, solvent_batch, params):
    x1 = _gcn_stack(solute_x, solute_edge_index, solute_batch, params, 'a')
    x2 = _gcn_stack(solvent_x, solvent_edge_index, solvent_batch, params, 'b')
    x = jnp.concatenate([x1, x2], axis=1)
    x = jax.nn.relu(x @ params['fc_W0'] + params['fc_b0'])
    x = _bn(x, params['fc_g0'], params['fc_be0'])
    x = jax.nn.relu(x @ params['fc_W1'] + params['fc_b1'])
    x = _bn(x, params['fc_g1'], params['fc_be1'])
    x = jax.nn.relu(x @ params['fc_W2'] + params['fc_b2'])
    x = x @ params['fc_W3'] + params['fc_b3']
    return pl.pallas_call(
        _copy_kernel,
        out_shape=jax.ShapeDtypeStruct(x.shape, x.dtype),
    )(x)


# speedup vs baseline: 9.7084x; 9.7084x over previous
import functools

import jax
import jax.numpy as jnp
from jax import lax
from jax.experimental import pallas as pl
from jax.experimental.pallas import tpu as pltpu
from jax.experimental.pallas import tpu_sc as plsc

N = 50000
E = 800000
NF = 75
D = 64
G = 128

NC = 2
NS = 16
NT = NC * NS
HALF = 25088
SL = 1568
PN = NC * HALF
ACC_R = 26624
ZR = 1664
DUMMY = HALF
CAP = 32768
C = 128
EC = 2000
ES = E // NS
ROWB = 28
RB = PN // ROWB
PR = 112
GP = 144

_mesh = plsc.VectorSubcoreMesh(core_axis_name="c", subcore_axis_name="s")


def _wid():
    return lax.axis_index("s") * NC + lax.axis_index("c")


def _zero_1d(ref, n):
    z = jnp.zeros((16,), ref.dtype)

    def body(i, _):
        ref[pl.ds(i * 16, 16)] = z
        return 0

    lax.fori_loop(0, n // 16, body, 0)


def _zero_2d(ref, nrows, ncols):
    z = jnp.zeros((16,), ref.dtype)

    def body(r, _):
        for c in range(ncols // 16):
            ref[r, pl.ds(c * 16, 16)] = z
        return 0

    lax.fori_loop(0, nrows, body, 0)


@functools.partial(
    pl.kernel,
    out_type=(
        jax.ShapeDtypeStruct((NT * CAP,), jnp.int32),
        jax.ShapeDtypeStruct((NT * CAP,), jnp.int32),
        jax.ShapeDtypeStruct((NT * 16,), jnp.int32),
        jax.ShapeDtypeStruct((PN,), jnp.float32),
    ),
    mesh=_mesh,
    compiler_params=pltpu.CompilerParams(needs_layout_passes=False, use_tc_tiling_on_sc=False),
    scratch_types=[
        pltpu.VMEM((EC,), jnp.int32),
        pltpu.VMEM((EC,), jnp.int32),
        pltpu.VMEM((CAP,), jnp.int32),
        pltpu.VMEM((CAP,), jnp.int32),
        pltpu.VMEM((C,), jnp.float32),
        pltpu.VMEM((C,), jnp.float32),
        pltpu.VMEM((C,), jnp.int32),
        pltpu.VMEM((16,), jnp.int32),
        pltpu.VMEM_SHARED((ACC_R,), jnp.float32),
    ],
)
def _partition_kernel(src_hbm, dst_hbm, slist_out, dlist_out, cnt_out, deg_out,
                      sbuf, dbuf, slist, dlist, ones, zeros, dstage, cstage,
                      degsh):
    cid = lax.axis_index("c")
    sid = lax.axis_index("s")
    wid = sid * NC + cid
    lo = cid * HALF
    ebase = sid * ES

    one = jnp.ones((16,), jnp.float32)
    zf = jnp.zeros((16,), jnp.float32)
    for j in range(C // 16):
        ones[pl.ds(j * 16, 16)] = one
        zeros[pl.ds(j * 16, 16)] = zf

    def zdeg(i, _):
        pltpu.sync_copy(zeros, degsh.at[pl.ds(sid * ZR + i * C, C)])
        return 0

    lax.fori_loop(0, ZR // C, zdeg, 0)

    def chunk_body(ci, off):
        pltpu.sync_copy(src_hbm.at[pl.ds(ebase + ci * EC, EC)], sbuf)
        pltpu.sync_copy(dst_hbm.at[pl.ds(ebase + ci * EC, EC)], dbuf)

        lane = lax.iota(jnp.int32, 16)

        def vec_body(v, off):
            sv = sbuf[pl.ds(v * 16, 16)]
            dv = dbuf[pl.ds(v * 16, 16)]
            m = (dv >= lo) & (dv < lo + HALF)
            pos = plsc.cumsum(m.astype(jnp.int32)) + (off - 1)
            tgt = jnp.where(m, pos, CAP - 16 + lane)
            plsc.store_scatter(slist, [tgt], sv)
            plsc.store_scatter(dlist, [tgt], dv - lo)
            return jnp.max(pos) + 1

        return lax.fori_loop(0, EC // 16, vec_body, off)

    off = lax.fori_loop(0, ES // EC, chunk_body, 0)

    zs = jnp.zeros((16,), jnp.int32)
    dum = jnp.full((16,), DUMMY, jnp.int32)
    for j in range(C // 16):
        slist[pl.ds(off + j * 16, 16)] = zs
        dlist[pl.ds(off + j * 16, 16)] = dum
    offp = ((off + C - 1) // C) * C

    plsc.subcore_barrier()

    def deg_body(j, _):
        for t in range(C // 16):
            dstage[pl.ds(t * 16, 16)] = dlist[pl.ds(j * C + t * 16, 16)]
        pltpu.sync_copy(ones, degsh.at[dstage], add=True)
        return 0

    lax.fori_loop(0, offp // C, deg_body, 0)

    cstage[pl.ds(0, 16)] = jnp.full((16,), offp, jnp.int32)
    pltpu.sync_copy(cstage, cnt_out.at[pl.ds(wid * 16, 16)])
    pltpu.sync_copy(slist, slist_out.at[pl.ds(wid * CAP, CAP)])
    pltpu.sync_copy(dlist, dlist_out.at[pl.ds(wid * CAP, CAP)])

    plsc.subcore_barrier()
    pltpu.sync_copy(degsh.at[pl.ds(sid * SL, SL)],
                    deg_out.at[pl.ds(cid * HALF + sid * SL, SL)])


@functools.partial(
    pl.kernel,
    out_type=jax.ShapeDtypeStruct((PN, D), jnp.float32),
    mesh=_mesh,
    compiler_params=pltpu.CompilerParams(needs_layout_passes=False, use_tc_tiling_on_sc=False),
    scratch_types=[
        pltpu.VMEM((C, D), jnp.float32),
        pltpu.VMEM((C,), jnp.int32),
        pltpu.VMEM((C,), jnp.int32),
        pltpu.VMEM((16,), jnp.int32),
        pltpu.SemaphoreType.DMA,
        pltpu.VMEM_SHARED((ACC_R, D), jnp.float32),
    ],
)
def _segsum_kernel(h_hbm, slist_hbm, dlist_hbm, cnt_hbm, s_out,
                   rows, sidx, didx, cstage, sem, sacc):
    cid = lax.axis_index("c")
    sid = lax.axis_index("s")
    wid = sid * NC + cid
    pltpu.sync_copy(cnt_hbm.at[pl.ds(wid * 16, 16)], cstage)
    k = jnp.max(cstage[pl.ds(0, 16)])

    _zero_2d(rows, C, D)

    def zacc(i, _):
        pltpu.sync_copy(rows, sacc.at[pl.ds(sid * ZR + i * C, C)])
        return 0

    lax.fori_loop(0, ZR // C, zacc, 0)
    plsc.subcore_barrier()

    base = wid * CAP

    def body(g, _):
        pltpu.sync_copy(slist_hbm.at[pl.ds(base + g * C, C)], sidx)
        pltpu.sync_copy(dlist_hbm.at[pl.ds(base + g * C, C)], didx)
        pltpu.async_copy(h_hbm.at[sidx], rows, sem).wait()
        pltpu.sync_copy(rows, sacc.at[didx], add=True)
        return 0

    lax.fori_loop(0, k // C, body, 0)

    plsc.subcore_barrier()
    pltpu.sync_copy(sacc.at[pl.ds(sid * SL, SL)],
                    s_out.at[pl.ds(cid * HALF + sid * SL, SL)])


@functools.partial(
    pl.kernel,
    out_type=(
        jax.ShapeDtypeStruct((NC, GP, D), jnp.float32),
        jax.ShapeDtypeStruct((NC, GP), jnp.float32),
    ),
    mesh=_mesh,
    compiler_params=pltpu.CompilerParams(needs_layout_passes=False, use_tc_tiling_on_sc=False),
    scratch_types=[
        pltpu.VMEM((PR, D), jnp.float32),
        pltpu.VMEM((PR,), jnp.int32),
        pltpu.VMEM((PR,), jnp.float32),
        pltpu.VMEM((GP,), jnp.float32),
        pltpu.VMEM_SHARED((GP, D), jnp.float32),
        pltpu.VMEM_SHARED((GP,), jnp.float32),
    ],
)
def _pool_kernel(r_hbm, batch_hbm, p_out, c_out,
                 rows, bidx, ones, zeros, shared, cshared):
    cid = lax.axis_index("c")
    sid = lax.axis_index("s")
    wid = sid * NC + cid
    base = wid * SL

    one = jnp.ones((16,), jnp.float32)
    for j in range(PR // 16):
        ones[pl.ds(j * 16, 16)] = one
    _zero_1d(zeros, GP)
    _zero_2d(rows, PR, D)

    @pl.when(sid == 0)
    def _():
        pltpu.sync_copy(zeros, cshared)
        pltpu.sync_copy(rows, shared.at[pl.ds(0, PR)])
        pltpu.sync_copy(rows, shared.at[pl.ds(GP - PR, PR)])

    plsc.subcore_barrier()

    def body(j, _):
        pltpu.sync_copy(batch_hbm.at[pl.ds(base + j * PR, PR)], bidx)
        pltpu.sync_copy(r_hbm.at[pl.ds(base + j * PR, PR)], rows)
        pltpu.sync_copy(rows, shared.at[bidx], add=True)
        pltpu.sync_copy(ones, cshared.at[bidx], add=True)
        return 0

    lax.fori_loop(0, SL // PR, body, 0)

    plsc.subcore_barrier()

    @pl.when(sid == 0)
    def _():
        pltpu.sync_copy(shared, p_out.at[cid])
        pltpu.sync_copy(cshared, c_out.at[cid])


def _k1_body(r_ref, w_ref, g_ref, be_ref, s1_ref, s2_ref, deg_ref, o_ref):
    s1 = s1_ref[...]
    s2 = s2_ref[...]
    m = s1 * (1.0 / N)
    v = s2 * (1.0 / N) - m * m
    xn = (r_ref[...] - m) / jnp.sqrt(v + 1e-5) * g_ref[...] + be_ref[...]
    h = jnp.dot(xn, w_ref[...], preferred_element_type=jnp.float32)
    dinv = lax.rsqrt(deg_ref[...] + 1.0)
    o_ref[...] = h * dinv


def _k1(r, w, g, be, s1, s2, deg):
    nf = w.shape[0]
    return pl.pallas_call(
        _k1_body,
        grid=(ROWB,),
        in_specs=[
            pl.BlockSpec((RB, nf), lambda i: (i, 0)),
            pl.BlockSpec((nf, D), lambda i: (0, 0)),
            pl.BlockSpec((1, nf), lambda i: (0, 0)),
            pl.BlockSpec((1, nf), lambda i: (0, 0)),
            pl.BlockSpec((1, nf), lambda i: (0, 0)),
            pl.BlockSpec((1, nf), lambda i: (0, 0)),
            pl.BlockSpec((RB, 1), lambda i: (i, 0)),
        ],
        out_specs=pl.BlockSpec((RB, D), lambda i: (i, 0)),
        out_shape=jax.ShapeDtypeStruct((PN, D), jnp.float32),
    )(r, w, g, be, s1, s2, deg)


def _k3_body(s_ref, h_ref, deg_ref, b_ref, r_ref, s1_ref, s2_ref):
    i = pl.program_id(0)
    dinv = lax.rsqrt(deg_ref[...] + 1.0)
    z = dinv * (s_ref[...] + h_ref[...]) + b_ref[...]
    r = jnp.maximum(z, 0.0)
    r_ref[...] = r
    rows = i * RB + lax.broadcasted_iota(jnp.int32, (RB, 1), 0)
    rm = jnp.where(rows < N, r, 0.0)
    ps1 = jnp.sum(rm, axis=0, keepdims=True)
    ps2 = jnp.sum(rm * rm, axis=0, keepdims=True)

    @pl.when(i == 0)
    def _():
        s1_ref[...] = ps1
        s2_ref[...] = ps2

    @pl.when(i > 0)
    def _():
        s1_ref[...] += ps1
        s2_ref[...] += ps2


def _k3(s, h, deg, b):
    return pl.pallas_call(
        _k3_body,
        grid=(ROWB,),
        in_specs=[
            pl.BlockSpec((RB, D), lambda i: (i, 0)),
            pl.BlockSpec((RB, D), lambda i: (i, 0)),
            pl.BlockSpec((RB, 1), lambda i: (i, 0)),
            pl.BlockSpec((1, D), lambda i: (0, 0)),
        ],
        out_specs=[
            pl.BlockSpec((RB, D), lambda i: (i, 0)),
            pl.BlockSpec((1, D), lambda i: (0, 0)),
            pl.BlockSpec((1, D), lambda i: (0, 0)),
        ],
        out_shape=[
            jax.ShapeDtypeStruct((PN, D), jnp.float32),
            jax.ShapeDtypeStruct((1, D), jnp.float32),
            jax.ShapeDtypeStruct((1, D), jnp.float32),
        ],
    )(s, h, deg, b)


def _bn_rows(x, g, b):
    m = jnp.mean(x, axis=0, keepdims=True)
    v = jnp.mean((x - m) * (x - m), axis=0, keepdims=True)
    return (x - m) / jnp.sqrt(v + 1e-5) * g + b


def _mlp_body(pa0_ref, pa1_ref, ca_ref, s1a_ref, s2a_ref, ga_ref, bea_ref,
              pb0_ref, pb1_ref, cb_ref, s1b_ref, s2b_ref, gb_ref, beb_ref,
              w0a_ref, w0b_ref, b0_ref, g0_ref, be0_ref,
              w1_ref, b1_ref, g1_ref, be1_ref,
              w2_ref, b2_ref, w3_ref, b3_ref, o_ref):
    def tower(p0_ref, p1_ref, c_ref, s1_ref, s2_ref, g_ref, be_ref):
        pooled = (p0_ref[...] + p1_ref[...])[:G]
        cnt = c_ref[...]
        m = s1_ref[...] * (1.0 / N)
        v = s2_ref[...] * (1.0 / N) - m * m
        rstd = lax.rsqrt(v + 1e-5)
        a = rstd * g_ref[...]
        b = be_ref[...] - m * a
        return pooled * a + cnt * b

    xa = tower(pa0_ref, pa1_ref, ca_ref, s1a_ref, s2a_ref, ga_ref, bea_ref)
    xb = tower(pb0_ref, pb1_ref, cb_ref, s1b_ref, s2b_ref, gb_ref, beb_ref)
    x = (jnp.dot(xa, w0a_ref[...], preferred_element_type=jnp.float32)
         + jnp.dot(xb, w0b_ref[...], preferred_element_type=jnp.float32)
         + b0_ref[...])
    x = jnp.maximum(x, 0.0)
    x = _bn_rows(x, g0_ref[...], be0_ref[...])
    x = jnp.maximum(jnp.dot(x, w1_ref[...], preferred_element_type=jnp.float32)
                    + b1_ref[...], 0.0)
    x = _bn_rows(x, g1_ref[...], be1_ref[...])
    x = jnp.maximum(jnp.dot(x, w2_ref[...], preferred_element_type=jnp.float32)
                    + b2_ref[...], 0.0)
    o_ref[...] = (jnp.dot(x, w3_ref[...], preferred_element_type=jnp.float32)
                  + b3_ref[...])


def _mlp(ta, tb, params):
    pa, ca, s1a, s2a, ga, bea = ta
    pb, cb, s1b, s2b, gb, beb = tb
    args = [pa[0], pa[1], _cnt_col(ca), s1a, s2a, ga, bea,
            pb[0], pb[1], _cnt_col(cb), s1b, s2b, gb, beb,
            params['fc_W0'][:D], params['fc_W0'][D:],
            params['fc_b0'].reshape(1, -1),
            params['fc_g0'].reshape(1, -1), params['fc_be0'].reshape(1, -1),
            params['fc_W1'], params['fc_b1'].reshape(1, -1),
            params['fc_g1'].reshape(1, -1), params['fc_be1'].reshape(1, -1),
            params['fc_W2'], params['fc_b2'].reshape(1, -1),
            params['fc_W3'], params['fc_b3'].reshape(1, -1)]
    return pl.pallas_call(
        _mlp_body,
        out_shape=jax.ShapeDtypeStruct((G, 1), jnp.float32),
    )(*args)


def _cnt_col(c_partials):
    return (c_partials[0] + c_partials[1])[:G].reshape(G, 1)


def _tower(x, edge_index, batch, params, pre):
    src = edge_index[0]
    dst = edge_index[1]
    slist, dlist, cnts, deg = _partition_kernel(src, dst)
    deg = deg.reshape(PN, 1)
    xp = jnp.pad(x, ((0, PN - N), (0, 0)))
    batch_p = jnp.pad(batch, (0, PN - N), constant_values=G)

    id_g = jnp.ones((1, NF), jnp.float32)
    id_be = jnp.zeros((1, NF), jnp.float32)
    id_s1 = jnp.zeros((1, NF), jnp.float32)
    id_s2 = jnp.full((1, NF), (1.0 - 1e-5) * N, jnp.float32)

    r, s1c, s2c = xp, id_s1, id_s2
    gc, bec = id_g, id_be
    s1 = s2 = None
    for i in range(4):
        w = params[pre + '_W' + str(i)]
        b = params[pre + '_b' + str(i)].reshape(1, D)
        h = _k1(r, w, gc, bec, s1c, s2c, deg)
        s = _segsum_kernel(h, slist, dlist, cnts)
        r, s1, s2 = _k3(s, h, deg, b)
        s1c, s2c = s1, s2
        gc = params[pre + '_g' + str(i)].reshape(1, D)
        bec = params[pre + '_be' + str(i)].reshape(1, D)

    p, c = _pool_kernel(r, batch_p)
    g_row = params[pre + '_g3'].reshape(1, D)
    be_row = params[pre + '_be3'].reshape(1, D)
    return p, c, s1, s2, g_row, be_row


def kernel(solute_x, solute_edge_index, solute_batch,
           solvent_x, solvent_edge_index, solvent_batch, params):
    ta = _tower(solute_x, solute_edge_index, solute_batch, params, 'a')
    tb = _tower(solvent_x, solvent_edge_index, solvent_batch, params, 'b')
    return _mlp(ta, tb, params)


# speedup vs baseline: 12.2556x; 1.2624x over previous
import functools

import jax
import jax.numpy as jnp
from jax import lax
from jax.experimental import pallas as pl
from jax.experimental.pallas import tpu as pltpu
from jax.experimental.pallas import tpu_sc as plsc

N = 50000
E = 800000
NF = 75
D = 64
G = 128

NC = 2
NS = 16
NT = NC * NS
HALF = 25088
SL = 1568
PN = NC * HALF
ACC_R = 25312
ZST = 1576
ZCH = 13
DUMMY = HALF
CAP = 32768
C = 128
NB = 3
SUP = NB * C
EC = 2000
ES = E // NS
ROWB = 28
RB = PN // ROWB
PR = 112
GP = 144

_mesh = plsc.VectorSubcoreMesh(core_axis_name="c", subcore_axis_name="s")


def _wid():
    return lax.axis_index("s") * NC + lax.axis_index("c")


def _zero_1d(ref, n):
    z = jnp.zeros((16,), ref.dtype)

    def body(i, _):
        ref[pl.ds(i * 16, 16)] = z
        return 0

    lax.fori_loop(0, n // 16, body, 0)


def _zero_2d(ref, nrows, ncols):
    z = jnp.zeros((16,), ref.dtype)

    def body(r, _):
        for c in range(ncols // 16):
            ref[r, pl.ds(c * 16, 16)] = z
        return 0

    lax.fori_loop(0, nrows, body, 0)


@functools.partial(
    pl.kernel,
    out_type=(
        jax.ShapeDtypeStruct((NT * CAP,), jnp.int32),
        jax.ShapeDtypeStruct((NT * CAP,), jnp.int32),
        jax.ShapeDtypeStruct((NT * 16,), jnp.int32),
        jax.ShapeDtypeStruct((PN,), jnp.float32),
    ),
    mesh=_mesh,
    compiler_params=pltpu.CompilerParams(needs_layout_passes=False, use_tc_tiling_on_sc=False),
    scratch_types=[
        pltpu.VMEM((EC,), jnp.int32),
        pltpu.VMEM((EC,), jnp.int32),
        pltpu.VMEM((CAP,), jnp.int32),
        pltpu.VMEM((CAP,), jnp.int32),
        pltpu.VMEM((C,), jnp.float32),
        pltpu.VMEM((C,), jnp.float32),
        pltpu.VMEM((NB, C), jnp.int32),
        pltpu.VMEM((16,), jnp.int32),
        pltpu.SemaphoreType.DMA,
        pltpu.VMEM_SHARED((ACC_R,), jnp.float32),
    ],
)
def _partition_kernel(src_hbm, dst_hbm, slist_out, dlist_out, cnt_out, deg_out,
                      sbuf, dbuf, slist, dlist, ones, zeros, dstage, cstage,
                      dsem, degsh):
    cid = lax.axis_index("c")
    sid = lax.axis_index("s")
    wid = sid * NC + cid
    lo = cid * HALF
    ebase = sid * ES

    one = jnp.ones((16,), jnp.float32)
    zf = jnp.zeros((16,), jnp.float32)
    for j in range(C // 16):
        ones[pl.ds(j * 16, 16)] = one
        zeros[pl.ds(j * 16, 16)] = zf

    def zdeg(i, _):
        pltpu.sync_copy(zeros, degsh.at[pl.ds(sid * ZST + i * C, C)])
        return 0

    lax.fori_loop(0, ZCH, zdeg, 0)

    def chunk_body(ci, off):
        pltpu.sync_copy(src_hbm.at[pl.ds(ebase + ci * EC, EC)], sbuf)
        pltpu.sync_copy(dst_hbm.at[pl.ds(ebase + ci * EC, EC)], dbuf)

        lane = lax.iota(jnp.int32, 16)

        def vec_body(v, off):
            sv = sbuf[pl.ds(v * 16, 16)]
            dv = dbuf[pl.ds(v * 16, 16)]
            m = (dv >= lo) & (dv < lo + HALF)
            pos = plsc.cumsum(m.astype(jnp.int32)) + (off - 1)
            tgt = jnp.where(m, pos, CAP - 16 + lane)
            plsc.store_scatter(slist, [tgt], sv)
            plsc.store_scatter(dlist, [tgt], dv - lo)
            return jnp.max(pos) + 1

        return lax.fori_loop(0, EC // 16, vec_body, off)

    off = lax.fori_loop(0, ES // EC, chunk_body, 0)

    zs = jnp.zeros((16,), jnp.int32)
    dum = jnp.full((16,), DUMMY, jnp.int32)
    for j in range(SUP // 16):
        slist[pl.ds(off + j * 16, 16)] = zs
        dlist[pl.ds(off + j * 16, 16)] = dum
    offp = ((off + SUP - 1) // SUP) * SUP

    plsc.subcore_barrier()

    def deg_body(js, _):
        descs = []
        for b in range(NB):
            for t in range(C // 16):
                dstage[b, pl.ds(t * 16, 16)] = dlist[pl.ds(js * SUP + b * C + t * 16, 16)]
            descs.append(pltpu.async_copy(ones, degsh.at[dstage.at[b]], dsem,
                                          add=True))
        for dsc in descs:
            dsc.wait()
        return 0

    lax.fori_loop(0, offp // SUP, deg_body, 0)

    cstage[pl.ds(0, 16)] = jnp.full((16,), offp, jnp.int32)
    pltpu.sync_copy(cstage, cnt_out.at[pl.ds(wid * 16, 16)])
    pltpu.sync_copy(slist, slist_out.at[pl.ds(wid * CAP, CAP)])
    pltpu.sync_copy(dlist, dlist_out.at[pl.ds(wid * CAP, CAP)])

    plsc.subcore_barrier()
    pltpu.sync_copy(degsh.at[pl.ds(sid * SL, SL)],
                    deg_out.at[pl.ds(cid * HALF + sid * SL, SL)])


@functools.partial(
    pl.kernel,
    out_type=jax.ShapeDtypeStruct((PN, D), jnp.float32),
    mesh=_mesh,
    compiler_params=pltpu.CompilerParams(needs_layout_passes=False, use_tc_tiling_on_sc=False),
    scratch_types=[
        pltpu.VMEM((NB, C, D), jnp.float32),
        pltpu.VMEM((NB, C), jnp.int32),
        pltpu.VMEM((NB, C), jnp.int32),
        pltpu.VMEM((16,), jnp.int32),
        pltpu.SemaphoreType.DMA,
        pltpu.SemaphoreType.DMA,
        pltpu.SemaphoreType.DMA,
        pltpu.SemaphoreType.DMA,
        pltpu.SemaphoreType.DMA,
        pltpu.SemaphoreType.DMA,
        pltpu.SemaphoreType.DMA,
        pltpu.SemaphoreType.DMA,
        pltpu.SemaphoreType.DMA,
        pltpu.VMEM_SHARED((ACC_R, D), jnp.float32),
    ],
)
def _segsum_kernel(h_hbm, slist_hbm, dlist_hbm, cnt_hbm, s_out,
                   rows, sidx, didx, cstage,
                   g0, g1, g2, g3, g4, g5, g6, g7, ssem, sacc):
    gsems = (g0, g1, g2, g3, g4, g5, g6, g7)
    cid = lax.axis_index("c")
    sid = lax.axis_index("s")
    wid = sid * NC + cid
    pltpu.sync_copy(cnt_hbm.at[pl.ds(wid * 16, 16)], cstage)
    k = jnp.max(cstage[pl.ds(0, 16)])

    _zero_2d(rows.at[0], C, D)

    def zacc(i, _):
        pltpu.sync_copy(rows.at[0], sacc.at[pl.ds(sid * ZST + i * C, C)])
        return 0

    lax.fori_loop(0, ZCH, zacc, 0)
    plsc.subcore_barrier()

    wrow = wid * (CAP // C)

    def sup(gi, _):
        rbase = wrow + gi * NB
        pltpu.sync_copy(slist_hbm.at[pl.ds(rbase, NB)], sidx)
        pltpu.sync_copy(dlist_hbm.at[pl.ds(rbase, NB)], didx)
        gds = []
        for b in range(NB):
            gds.append(pltpu.async_copy(h_hbm.at[sidx.at[b]], rows.at[b],
                                        gsems[b]))
        sds = []
        for b in range(NB):
            gds[b].wait()
            sds.append(pltpu.async_copy(rows.at[b], sacc.at[didx.at[b]], ssem,
                                        add=True))
        for dsc in sds:
            dsc.wait()
        return 0

    lax.fori_loop(0, k // SUP, sup, 0)

    plsc.subcore_barrier()
    pltpu.sync_copy(sacc.at[pl.ds(sid * SL, SL)],
                    s_out.at[pl.ds(cid * HALF + sid * SL, SL)])


@functools.partial(
    pl.kernel,
    out_type=(
        jax.ShapeDtypeStruct((NC, GP, D), jnp.float32),
        jax.ShapeDtypeStruct((NC, GP), jnp.float32),
    ),
    mesh=_mesh,
    compiler_params=pltpu.CompilerParams(needs_layout_passes=False, use_tc_tiling_on_sc=False),
    scratch_types=[
        pltpu.VMEM((PR, D), jnp.float32),
        pltpu.VMEM((PR,), jnp.int32),
        pltpu.VMEM((PR,), jnp.float32),
        pltpu.VMEM((GP,), jnp.float32),
        pltpu.VMEM_SHARED((GP, D), jnp.float32),
        pltpu.VMEM_SHARED((GP,), jnp.float32),
    ],
)
def _pool_kernel(r_hbm, batch_hbm, p_out, c_out,
                 rows, bidx, ones, zeros, shared, cshared):
    cid = lax.axis_index("c")
    sid = lax.axis_index("s")
    wid = sid * NC + cid
    base = wid * SL

    one = jnp.ones((16,), jnp.float32)
    for j in range(PR // 16):
        ones[pl.ds(j * 16, 16)] = one
    _zero_1d(zeros, GP)
    _zero_2d(rows, PR, D)

    @pl.when(sid == 0)
    def _():
        pltpu.sync_copy(zeros, cshared)
        pltpu.sync_copy(rows, shared.at[pl.ds(0, PR)])
        pltpu.sync_copy(rows, shared.at[pl.ds(GP - PR, PR)])

    plsc.subcore_barrier()

    def body(j, _):
        pltpu.sync_copy(batch_hbm.at[pl.ds(base + j * PR, PR)], bidx)
        pltpu.sync_copy(r_hbm.at[pl.ds(base + j * PR, PR)], rows)
        pltpu.sync_copy(rows, shared.at[bidx], add=True)
        pltpu.sync_copy(ones, cshared.at[bidx], add=True)
        return 0

    lax.fori_loop(0, SL // PR, body, 0)

    plsc.subcore_barrier()

    @pl.when(sid == 0)
    def _():
        pltpu.sync_copy(shared, p_out.at[cid])
        pltpu.sync_copy(cshared, c_out.at[cid])


def _k1_body(r_ref, w_ref, g_ref, be_ref, s1_ref, s2_ref, deg_ref, o_ref):
    s1 = s1_ref[...]
    s2 = s2_ref[...]
    m = s1 * (1.0 / N)
    v = s2 * (1.0 / N) - m * m
    xn = (r_ref[...] - m) / jnp.sqrt(v + 1e-5) * g_ref[...] + be_ref[...]
    h = jnp.dot(xn, w_ref[...], preferred_element_type=jnp.float32)
    dinv = lax.rsqrt(deg_ref[...] + 1.0)
    o_ref[...] = h * dinv


def _k1(r, w, g, be, s1, s2, deg):
    nf = w.shape[0]
    return pl.pallas_call(
        _k1_body,
        grid=(ROWB,),
        in_specs=[
            pl.BlockSpec((RB, nf), lambda i: (i, 0)),
            pl.BlockSpec((nf, D), lambda i: (0, 0)),
            pl.BlockSpec((1, nf), lambda i: (0, 0)),
            pl.BlockSpec((1, nf), lambda i: (0, 0)),
            pl.BlockSpec((1, nf), lambda i: (0, 0)),
            pl.BlockSpec((1, nf), lambda i: (0, 0)),
            pl.BlockSpec((RB, 1), lambda i: (i, 0)),
        ],
        out_specs=pl.BlockSpec((RB, D), lambda i: (i, 0)),
        out_shape=jax.ShapeDtypeStruct((PN, D), jnp.float32),
    )(r, w, g, be, s1, s2, deg)


def _k3_body(s_ref, h_ref, deg_ref, b_ref, r_ref, s1_ref, s2_ref):
    i = pl.program_id(0)
    dinv = lax.rsqrt(deg_ref[...] + 1.0)
    z = dinv * (s_ref[...] + h_ref[...]) + b_ref[...]
    r = jnp.maximum(z, 0.0)
    r_ref[...] = r
    rows = i * RB + lax.broadcasted_iota(jnp.int32, (RB, 1), 0)
    rm = jnp.where(rows < N, r, 0.0)
    ps1 = jnp.sum(rm, axis=0, keepdims=True)
    ps2 = jnp.sum(rm * rm, axis=0, keepdims=True)

    @pl.when(i == 0)
    def _():
        s1_ref[...] = ps1
        s2_ref[...] = ps2

    @pl.when(i > 0)
    def _():
        s1_ref[...] += ps1
        s2_ref[...] += ps2


def _k3(s, h, deg, b):
    return pl.pallas_call(
        _k3_body,
        grid=(ROWB,),
        in_specs=[
            pl.BlockSpec((RB, D), lambda i: (i, 0)),
            pl.BlockSpec((RB, D), lambda i: (i, 0)),
            pl.BlockSpec((RB, 1), lambda i: (i, 0)),
            pl.BlockSpec((1, D), lambda i: (0, 0)),
        ],
        out_specs=[
            pl.BlockSpec((RB, D), lambda i: (i, 0)),
            pl.BlockSpec((1, D), lambda i: (0, 0)),
            pl.BlockSpec((1, D), lambda i: (0, 0)),
        ],
        out_shape=[
            jax.ShapeDtypeStruct((PN, D), jnp.float32),
            jax.ShapeDtypeStruct((1, D), jnp.float32),
            jax.ShapeDtypeStruct((1, D), jnp.float32),
        ],
    )(s, h, deg, b)


def _bn_rows(x, g, b):
    m = jnp.mean(x, axis=0, keepdims=True)
    v = jnp.mean((x - m) * (x - m), axis=0, keepdims=True)
    return (x - m) / jnp.sqrt(v + 1e-5) * g + b


def _mlp_body(pa0_ref, pa1_ref, ca_ref, s1a_ref, s2a_ref, ga_ref, bea_ref,
              pb0_ref, pb1_ref, cb_ref, s1b_ref, s2b_ref, gb_ref, beb_ref,
              w0a_ref, w0b_ref, b0_ref, g0_ref, be0_ref,
              w1_ref, b1_ref, g1_ref, be1_ref,
              w2_ref, b2_ref, w3_ref, b3_ref, o_ref):
    def tower(p0_ref, p1_ref, c_ref, s1_ref, s2_ref, g_ref, be_ref):
        pooled = (p0_ref[...] + p1_ref[...])[:G]
        cnt = c_ref[...]
        m = s1_ref[...] * (1.0 / N)
        v = s2_ref[...] * (1.0 / N) - m * m
        rstd = lax.rsqrt(v + 1e-5)
        a = rstd * g_ref[...]
        b = be_ref[...] - m * a
        return pooled * a + cnt * b

    xa = tower(pa0_ref, pa1_ref, ca_ref, s1a_ref, s2a_ref, ga_ref, bea_ref)
    xb = tower(pb0_ref, pb1_ref, cb_ref, s1b_ref, s2b_ref, gb_ref, beb_ref)
    x = (jnp.dot(xa, w0a_ref[...], preferred_element_type=jnp.float32)
         + jnp.dot(xb, w0b_ref[...], preferred_element_type=jnp.float32)
         + b0_ref[...])
    x = jnp.maximum(x, 0.0)
    x = _bn_rows(x, g0_ref[...], be0_ref[...])
    x = jnp.maximum(jnp.dot(x, w1_ref[...], preferred_element_type=jnp.float32)
                    + b1_ref[...], 0.0)
    x = _bn_rows(x, g1_ref[...], be1_ref[...])
    x = jnp.maximum(jnp.dot(x, w2_ref[...], preferred_element_type=jnp.float32)
                    + b2_ref[...], 0.0)
    o_ref[...] = (jnp.dot(x, w3_ref[...], preferred_element_type=jnp.float32)
                  + b3_ref[...])


def _mlp(ta, tb, params):
    pa, ca, s1a, s2a, ga, bea = ta
    pb, cb, s1b, s2b, gb, beb = tb
    args = [pa[0], pa[1], _cnt_col(ca), s1a, s2a, ga, bea,
            pb[0], pb[1], _cnt_col(cb), s1b, s2b, gb, beb,
            params['fc_W0'][:D], params['fc_W0'][D:],
            params['fc_b0'].reshape(1, -1),
            params['fc_g0'].reshape(1, -1), params['fc_be0'].reshape(1, -1),
            params['fc_W1'], params['fc_b1'].reshape(1, -1),
            params['fc_g1'].reshape(1, -1), params['fc_be1'].reshape(1, -1),
            params['fc_W2'], params['fc_b2'].reshape(1, -1),
            params['fc_W3'], params['fc_b3'].reshape(1, -1)]
    return pl.pallas_call(
        _mlp_body,
        out_shape=jax.ShapeDtypeStruct((G, 1), jnp.float32),
    )(*args)


def _cnt_col(c_partials):
    return (c_partials[0] + c_partials[1])[:G].reshape(G, 1)


def _tower(x, edge_index, batch, params, pre):
    src = edge_index[0]
    dst = edge_index[1]
    slist, dlist, cnts, deg = _partition_kernel(src, dst)
    slist = slist.reshape(NT * CAP // C, C)
    dlist = dlist.reshape(NT * CAP // C, C)
    deg = deg.reshape(PN, 1)
    xp = jnp.pad(x, ((0, PN - N), (0, 0)))
    batch_p = jnp.pad(batch, (0, PN - N), constant_values=G)

    id_g = jnp.ones((1, NF), jnp.float32)
    id_be = jnp.zeros((1, NF), jnp.float32)
    id_s1 = jnp.zeros((1, NF), jnp.float32)
    id_s2 = jnp.full((1, NF), (1.0 - 1e-5) * N, jnp.float32)

    r, s1c, s2c = xp, id_s1, id_s2
    gc, bec = id_g, id_be
    s1 = s2 = None
    for i in range(4):
        w = params[pre + '_W' + str(i)]
        b = params[pre + '_b' + str(i)].reshape(1, D)
        h = _k1(r, w, gc, bec, s1c, s2c, deg)
        s = _segsum_kernel(h, slist, dlist, cnts)
        r, s1, s2 = _k3(s, h, deg, b)
        s1c, s2c = s1, s2
        gc = params[pre + '_g' + str(i)].reshape(1, D)
        bec = params[pre + '_be' + str(i)].reshape(1, D)

    p, c = _pool_kernel(r, batch_p)
    g_row = params[pre + '_g3'].reshape(1, D)
    be_row = params[pre + '_be3'].reshape(1, D)
    return p, c, s1, s2, g_row, be_row


def kernel(solute_x, solute_edge_index, solute_batch,
           solvent_x, solvent_edge_index, solvent_batch, params):
    ta = _tower(solute_x, solute_edge_index, solute_batch, params, 'a')
    tb = _tower(solvent_x, solvent_edge_index, solvent_batch, params, 'b')
    return _mlp(ta, tb, params)


# speedup vs baseline: 12.2582x; 1.0002x over previous
import functools

import jax
import jax.numpy as jnp
from jax import lax
from jax.experimental import pallas as pl
from jax.experimental.pallas import tpu as pltpu
from jax.experimental.pallas import tpu_sc as plsc

N = 50000
E = 800000
NF = 75
D = 64
G = 128

NC = 2
NS = 16
NT = NC * NS
HALF = 25088
SL = 1568
PN = NC * HALF
ACC_R = 25312
ZST = 1576
ZCH = 13
DUMMY = HALF
CAP = 32768
C = 128
NB = 3
SUP = NB * C
EC = 2000
ES = E // NS
ROWB = 28
RB = PN // ROWB
PR = 112
GP = 144

_mesh = plsc.VectorSubcoreMesh(core_axis_name="c", subcore_axis_name="s")


def _wid():
    return lax.axis_index("s") * NC + lax.axis_index("c")


def _zero_1d(ref, n):
    z = jnp.zeros((16,), ref.dtype)

    def body(i, _):
        ref[pl.ds(i * 16, 16)] = z
        return 0

    lax.fori_loop(0, n // 16, body, 0)


def _zero_2d(ref, nrows, ncols):
    z = jnp.zeros((16,), ref.dtype)

    def body(r, _):
        for c in range(ncols // 16):
            ref[r, pl.ds(c * 16, 16)] = z
        return 0

    lax.fori_loop(0, nrows, body, 0)


@functools.partial(
    pl.kernel,
    out_type=(
        jax.ShapeDtypeStruct((NT * CAP,), jnp.int32),
        jax.ShapeDtypeStruct((NT * CAP,), jnp.int32),
        jax.ShapeDtypeStruct((NT * 16,), jnp.int32),
        jax.ShapeDtypeStruct((PN,), jnp.float32),
    ),
    mesh=_mesh,
    compiler_params=pltpu.CompilerParams(needs_layout_passes=False, use_tc_tiling_on_sc=False),
    scratch_types=[
        pltpu.VMEM((EC,), jnp.int32),
        pltpu.VMEM((EC,), jnp.int32),
        pltpu.VMEM((CAP,), jnp.int32),
        pltpu.VMEM((CAP,), jnp.int32),
        pltpu.VMEM((C,), jnp.float32),
        pltpu.VMEM((C,), jnp.float32),
        pltpu.VMEM((NB, C), jnp.int32),
        pltpu.VMEM((16,), jnp.int32),
        pltpu.SemaphoreType.DMA,
        pltpu.VMEM_SHARED((ACC_R,), jnp.float32),
    ],
)
def _partition_kernel(src_hbm, dst_hbm, slist_out, dlist_out, cnt_out, deg_out,
                      sbuf, dbuf, slist, dlist, ones, zeros, dstage, cstage,
                      dsem, degsh):
    cid = lax.axis_index("c")
    sid = lax.axis_index("s")
    wid = sid * NC + cid
    lo = cid * HALF
    ebase = sid * ES

    one = jnp.ones((16,), jnp.float32)
    zf = jnp.zeros((16,), jnp.float32)
    for j in range(C // 16):
        ones[pl.ds(j * 16, 16)] = one
        zeros[pl.ds(j * 16, 16)] = zf

    def zdeg(i, _):
        pltpu.sync_copy(zeros, degsh.at[pl.ds(sid * ZST + i * C, C)])
        return 0

    lax.fori_loop(0, ZCH, zdeg, 0)

    def chunk_body(ci, off):
        pltpu.sync_copy(src_hbm.at[pl.ds(ebase + ci * EC, EC)], sbuf)
        pltpu.sync_copy(dst_hbm.at[pl.ds(ebase + ci * EC, EC)], dbuf)

        lane = lax.iota(jnp.int32, 16)

        def vec_body(v, off):
            sv = sbuf[pl.ds(v * 16, 16)]
            dv = dbuf[pl.ds(v * 16, 16)]
            m = (dv >= lo) & (dv < lo + HALF)
            pos = plsc.cumsum(m.astype(jnp.int32)) + (off - 1)
            tgt = jnp.where(m, pos, CAP - 16 + lane)
            plsc.store_scatter(slist, [tgt], sv)
            plsc.store_scatter(dlist, [tgt], dv - lo)
            return jnp.max(pos) + 1

        return lax.fori_loop(0, EC // 16, vec_body, off)

    off = lax.fori_loop(0, ES // EC, chunk_body, 0)

    zs = jnp.zeros((16,), jnp.int32)
    dum = jnp.full((16,), DUMMY, jnp.int32)
    for j in range(SUP // 16):
        slist[pl.ds(off + j * 16, 16)] = zs
        dlist[pl.ds(off + j * 16, 16)] = dum
    offp = ((off + SUP - 1) // SUP) * SUP

    plsc.subcore_barrier()

    def deg_body(js, _):
        descs = []
        for b in range(NB):
            for t in range(C // 16):
                dstage[b, pl.ds(t * 16, 16)] = dlist[pl.ds(js * SUP + b * C + t * 16, 16)]
            descs.append(pltpu.async_copy(ones, degsh.at[dstage.at[b]], dsem,
                                          add=True))
        for dsc in descs:
            dsc.wait()
        return 0

    lax.fori_loop(0, offp // SUP, deg_body, 0)

    cstage[pl.ds(0, 16)] = jnp.full((16,), offp, jnp.int32)
    pltpu.sync_copy(cstage, cnt_out.at[pl.ds(wid * 16, 16)])
    pltpu.sync_copy(slist, slist_out.at[pl.ds(wid * CAP, CAP)])
    pltpu.sync_copy(dlist, dlist_out.at[pl.ds(wid * CAP, CAP)])

    plsc.subcore_barrier()
    pltpu.sync_copy(degsh.at[pl.ds(sid * SL, SL)],
                    deg_out.at[pl.ds(cid * HALF + sid * SL, SL)])


@functools.partial(
    pl.kernel,
    out_type=jax.ShapeDtypeStruct((PN, D), jnp.float32),
    mesh=_mesh,
    compiler_params=pltpu.CompilerParams(needs_layout_passes=False, use_tc_tiling_on_sc=False),
    scratch_types=[
        pltpu.VMEM((NB, C, D), jnp.float32),
        pltpu.VMEM((NB, C), jnp.int32),
        pltpu.VMEM((NB, C), jnp.int32),
        pltpu.VMEM((16,), jnp.int32),
        pltpu.SemaphoreType.DMA,
        pltpu.SemaphoreType.DMA,
        pltpu.SemaphoreType.DMA,
        pltpu.SemaphoreType.DMA,
        pltpu.SemaphoreType.DMA,
        pltpu.SemaphoreType.DMA,
        pltpu.SemaphoreType.DMA,
        pltpu.SemaphoreType.DMA,
        pltpu.SemaphoreType.DMA,
        pltpu.VMEM_SHARED((ACC_R, D), jnp.float32),
    ],
)
def _segsum_kernel(h_hbm, slist_hbm, dlist_hbm, cnt_hbm, s_out,
                   rows, sidx, didx, cstage,
                   g0, g1, g2, g3, g4, g5, g6, g7, ssem, sacc):
    gsems = (g0, g1, g2, g3, g4, g5, g6, g7)
    cid = lax.axis_index("c")
    sid = lax.axis_index("s")
    wid = sid * NC + cid
    pltpu.sync_copy(cnt_hbm.at[pl.ds(wid * 16, 16)], cstage)
    k = jnp.max(cstage[pl.ds(0, 16)])

    _zero_2d(rows.at[0], C, D)

    def zacc(i, _):
        pltpu.sync_copy(rows.at[0], sacc.at[pl.ds(sid * ZST + i * C, C)])
        return 0

    lax.fori_loop(0, ZCH, zacc, 0)
    plsc.subcore_barrier()

    wrow = wid * (CAP // C)

    nsup = k // SUP

    def sup(gi, _):
        rbase = wrow + gi * NB

        @pl.when(gi > 0)
        def _():
            for b in range(NB):
                pltpu.make_async_copy(h_hbm.at[pl.ds(0, C)], rows.at[b],
                                      ssem).wait()

        pltpu.sync_copy(slist_hbm.at[pl.ds(rbase, NB)], sidx)
        pltpu.sync_copy(dlist_hbm.at[pl.ds(rbase, NB)], didx)
        gds = []
        for b in range(NB):
            gds.append(pltpu.async_copy(h_hbm.at[sidx.at[b]], rows.at[b],
                                        gsems[b]))
        for b in range(NB):
            gds[b].wait()
            pltpu.async_copy(rows.at[b], sacc.at[didx.at[b]], ssem, add=True)
        return 0

    lax.fori_loop(0, nsup, sup, 0)

    @pl.when(nsup > 0)
    def _():
        for b in range(NB):
            pltpu.make_async_copy(h_hbm.at[pl.ds(0, C)], rows.at[b],
                                  ssem).wait()

    plsc.subcore_barrier()
    pltpu.sync_copy(sacc.at[pl.ds(sid * SL, SL)],
                    s_out.at[pl.ds(cid * HALF + sid * SL, SL)])


@functools.partial(
    pl.kernel,
    out_type=(
        jax.ShapeDtypeStruct((NC, GP, D), jnp.float32),
        jax.ShapeDtypeStruct((NC, GP), jnp.float32),
    ),
    mesh=_mesh,
    compiler_params=pltpu.CompilerParams(needs_layout_passes=False, use_tc_tiling_on_sc=False),
    scratch_types=[
        pltpu.VMEM((PR, D), jnp.float32),
        pltpu.VMEM((PR,), jnp.int32),
        pltpu.VMEM((PR,), jnp.float32),
        pltpu.VMEM((GP,), jnp.float32),
        pltpu.VMEM_SHARED((GP, D), jnp.float32),
        pltpu.VMEM_SHARED((GP,), jnp.float32),
    ],
)
def _pool_kernel(r_hbm, batch_hbm, p_out, c_out,
                 rows, bidx, ones, zeros, shared, cshared):
    cid = lax.axis_index("c")
    sid = lax.axis_index("s")
    wid = sid * NC + cid
    base = wid * SL

    one = jnp.ones((16,), jnp.float32)
    for j in range(PR // 16):
        ones[pl.ds(j * 16, 16)] = one
    _zero_1d(zeros, GP)
    _zero_2d(rows, PR, D)

    @pl.when(sid == 0)
    def _():
        pltpu.sync_copy(zeros, cshared)
        pltpu.sync_copy(rows, shared.at[pl.ds(0, PR)])
        pltpu.sync_copy(rows, shared.at[pl.ds(GP - PR, PR)])

    plsc.subcore_barrier()

    def body(j, _):
        pltpu.sync_copy(batch_hbm.at[pl.ds(base + j * PR, PR)], bidx)
        pltpu.sync_copy(r_hbm.at[pl.ds(base + j * PR, PR)], rows)
        pltpu.sync_copy(rows, shared.at[bidx], add=True)
        pltpu.sync_copy(ones, cshared.at[bidx], add=True)
        return 0

    lax.fori_loop(0, SL // PR, body, 0)

    plsc.subcore_barrier()

    @pl.when(sid == 0)
    def _():
        pltpu.sync_copy(shared, p_out.at[cid])
        pltpu.sync_copy(cshared, c_out.at[cid])


def _k1_body(r_ref, w_ref, g_ref, be_ref, s1_ref, s2_ref, deg_ref, o_ref):
    s1 = s1_ref[...]
    s2 = s2_ref[...]
    m = s1 * (1.0 / N)
    v = s2 * (1.0 / N) - m * m
    xn = (r_ref[...] - m) / jnp.sqrt(v + 1e-5) * g_ref[...] + be_ref[...]
    h = jnp.dot(xn, w_ref[...], preferred_element_type=jnp.float32)
    dinv = lax.rsqrt(deg_ref[...] + 1.0)
    o_ref[...] = h * dinv


def _k1(r, w, g, be, s1, s2, deg):
    nf = w.shape[0]
    return pl.pallas_call(
        _k1_body,
        grid=(ROWB,),
        in_specs=[
            pl.BlockSpec((RB, nf), lambda i: (i, 0)),
            pl.BlockSpec((nf, D), lambda i: (0, 0)),
            pl.BlockSpec((1, nf), lambda i: (0, 0)),
            pl.BlockSpec((1, nf), lambda i: (0, 0)),
            pl.BlockSpec((1, nf), lambda i: (0, 0)),
            pl.BlockSpec((1, nf), lambda i: (0, 0)),
            pl.BlockSpec((RB, 1), lambda i: (i, 0)),
        ],
        out_specs=pl.BlockSpec((RB, D), lambda i: (i, 0)),
        out_shape=jax.ShapeDtypeStruct((PN, D), jnp.float32),
    )(r, w, g, be, s1, s2, deg)


def _k3_body(s_ref, h_ref, deg_ref, b_ref, r_ref, s1_ref, s2_ref):
    i = pl.program_id(0)
    dinv = lax.rsqrt(deg_ref[...] + 1.0)
    z = dinv * (s_ref[...] + h_ref[...]) + b_ref[...]
    r = jnp.maximum(z, 0.0)
    r_ref[...] = r
    rows = i * RB + lax.broadcasted_iota(jnp.int32, (RB, 1), 0)
    rm = jnp.where(rows < N, r, 0.0)
    ps1 = jnp.sum(rm, axis=0, keepdims=True)
    ps2 = jnp.sum(rm * rm, axis=0, keepdims=True)

    @pl.when(i == 0)
    def _():
        s1_ref[...] = ps1
        s2_ref[...] = ps2

    @pl.when(i > 0)
    def _():
        s1_ref[...] += ps1
        s2_ref[...] += ps2


def _k3(s, h, deg, b):
    return pl.pallas_call(
        _k3_body,
        grid=(ROWB,),
        in_specs=[
            pl.BlockSpec((RB, D), lambda i: (i, 0)),
            pl.BlockSpec((RB, D), lambda i: (i, 0)),
            pl.BlockSpec((RB, 1), lambda i: (i, 0)),
            pl.BlockSpec((1, D), lambda i: (0, 0)),
        ],
        out_specs=[
            pl.BlockSpec((RB, D), lambda i: (i, 0)),
            pl.BlockSpec((1, D), lambda i: (0, 0)),
            pl.BlockSpec((1, D), lambda i: (0, 0)),
        ],
        out_shape=[
            jax.ShapeDtypeStruct((PN, D), jnp.float32),
            jax.ShapeDtypeStruct((1, D), jnp.float32),
            jax.ShapeDtypeStruct((1, D), jnp.float32),
        ],
    )(s, h, deg, b)


def _bn_rows(x, g, b):
    m = jnp.mean(x, axis=0, keepdims=True)
    v = jnp.mean((x - m) * (x - m), axis=0, keepdims=True)
    return (x - m) / jnp.sqrt(v + 1e-5) * g + b


def _mlp_body(pa0_ref, pa1_ref, ca_ref, s1a_ref, s2a_ref, ga_ref, bea_ref,
              pb0_ref, pb1_ref, cb_ref, s1b_ref, s2b_ref, gb_ref, beb_ref,
              w0a_ref, w0b_ref, b0_ref, g0_ref, be0_ref,
              w1_ref, b1_ref, g1_ref, be1_ref,
              w2_ref, b2_ref, w3_ref, b3_ref, o_ref):
    def tower(p0_ref, p1_ref, c_ref, s1_ref, s2_ref, g_ref, be_ref):
        pooled = (p0_ref[...] + p1_ref[...])[:G]
        cnt = c_ref[...]
        m = s1_ref[...] * (1.0 / N)
        v = s2_ref[...] * (1.0 / N) - m * m
        rstd = lax.rsqrt(v + 1e-5)
        a = rstd * g_ref[...]
        b = be_ref[...] - m * a
        return pooled * a + cnt * b

    xa = tower(pa0_ref, pa1_ref, ca_ref, s1a_ref, s2a_ref, ga_ref, bea_ref)
    xb = tower(pb0_ref, pb1_ref, cb_ref, s1b_ref, s2b_ref, gb_ref, beb_ref)
    x = (jnp.dot(xa, w0a_ref[...], preferred_element_type=jnp.float32)
         + jnp.dot(xb, w0b_ref[...], preferred_element_type=jnp.float32)
         + b0_ref[...])
    x = jnp.maximum(x, 0.0)
    x = _bn_rows(x, g0_ref[...], be0_ref[...])
    x = jnp.maximum(jnp.dot(x, w1_ref[...], preferred_element_type=jnp.float32)
                    + b1_ref[...], 0.0)
    x = _bn_rows(x, g1_ref[...], be1_ref[...])
    x = jnp.maximum(jnp.dot(x, w2_ref[...], preferred_element_type=jnp.float32)
                    + b2_ref[...], 0.0)
    o_ref[...] = (jnp.dot(x, w3_ref[...], preferred_element_type=jnp.float32)
                  + b3_ref[...])


def _mlp(ta, tb, params):
    pa, ca, s1a, s2a, ga, bea = ta
    pb, cb, s1b, s2b, gb, beb = tb
    args = [pa[0], pa[1], _cnt_col(ca), s1a, s2a, ga, bea,
            pb[0], pb[1], _cnt_col(cb), s1b, s2b, gb, beb,
            params['fc_W0'][:D], params['fc_W0'][D:],
            params['fc_b0'].reshape(1, -1),
            params['fc_g0'].reshape(1, -1), params['fc_be0'].reshape(1, -1),
            params['fc_W1'], params['fc_b1'].reshape(1, -1),
            params['fc_g1'].reshape(1, -1), params['fc_be1'].reshape(1, -1),
            params['fc_W2'], params['fc_b2'].reshape(1, -1),
            params['fc_W3'], params['fc_b3'].reshape(1, -1)]
    return pl.pallas_call(
        _mlp_body,
        out_shape=jax.ShapeDtypeStruct((G, 1), jnp.float32),
    )(*args)


def _cnt_col(c_partials):
    return (c_partials[0] + c_partials[1])[:G].reshape(G, 1)


def _tower(x, edge_index, batch, params, pre):
    src = edge_index[0]
    dst = edge_index[1]
    slist, dlist, cnts, deg = _partition_kernel(src, dst)
    slist = slist.reshape(NT * CAP // C, C)
    dlist = dlist.reshape(NT * CAP // C, C)
    deg = deg.reshape(PN, 1)
    xp = jnp.pad(x, ((0, PN - N), (0, 0)))
    batch_p = jnp.pad(batch, (0, PN - N), constant_values=G)

    id_g = jnp.ones((1, NF), jnp.float32)
    id_be = jnp.zeros((1, NF), jnp.float32)
    id_s1 = jnp.zeros((1, NF), jnp.float32)
    id_s2 = jnp.full((1, NF), (1.0 - 1e-5) * N, jnp.float32)

    r, s1c, s2c = xp, id_s1, id_s2
    gc, bec = id_g, id_be
    s1 = s2 = None
    for i in range(4):
        w = params[pre + '_W' + str(i)]
        b = params[pre + '_b' + str(i)].reshape(1, D)
        h = _k1(r, w, gc, bec, s1c, s2c, deg)
        s = _segsum_kernel(h, slist, dlist, cnts)
        r, s1, s2 = _k3(s, h, deg, b)
        s1c, s2c = s1, s2
        gc = params[pre + '_g' + str(i)].reshape(1, D)
        bec = params[pre + '_be' + str(i)].reshape(1, D)

    p, c = _pool_kernel(r, batch_p)
    g_row = params[pre + '_g3'].reshape(1, D)
    be_row = params[pre + '_be3'].reshape(1, D)
    return p, c, s1, s2, g_row, be_row


def kernel(solute_x, solute_edge_index, solute_batch,
           solvent_x, solvent_edge_index, solvent_batch, params):
    ta = _tower(solute_x, solute_edge_index, solute_batch, params, 'a')
    tb = _tower(solvent_x, solvent_edge_index, solvent_batch, params, 'b')
    return _mlp(ta, tb, params)
